# baseline (speedup 1.0000x reference)
import functools

import jax
import jax.numpy as jnp
import numpy as np
from jax import lax
from jax.experimental import pallas as pl
from jax.experimental.pallas import tpu as pltpu
from jax.experimental.pallas import tpu_sc as plsc

T, D, H, E, K = 2048, 1024, 512, 8, 2
BLK = 128
NB = T * K // BLK + E
R = NB * BLK
NW = 32
PAIRS_PER_W = T * K // NW
CH = 32
NSH = PAIRS_PER_W // CH
TPW = T // NW
TPS = CH // K


def _router_body(x_ref, wg_ref, dest_ref, gexp_ref, be_ref, xb_ref):
    x = x_ref[...]
    xb_ref[...] = x.astype(jnp.bfloat16)
    logits = jnp.dot(x, wg_ref[...], preferred_element_type=jnp.float32)
    m = jnp.max(logits, axis=-1, keepdims=True)
    ex = jnp.exp(logits - m)
    p = ex / jnp.sum(ex, axis=-1, keepdims=True)
    lane = lax.broadcasted_iota(jnp.int32, (T, E), 1)
    v1 = jnp.max(p, axis=-1, keepdims=True)
    i1 = jnp.min(jnp.where(p == v1, lane, E), axis=-1, keepdims=True)
    p2 = jnp.where(lane == i1, -jnp.inf, p)
    v2 = jnp.max(p2, axis=-1, keepdims=True)
    i2 = jnp.min(jnp.where(p2 == v2, lane, E), axis=-1, keepdims=True)
    oh1 = (lane == i1).astype(jnp.float32)
    oh2 = (lane == i2).astype(jnp.float32)
    c = oh1 + oh2
    rowi = lax.broadcasted_iota(jnp.int32, (T, T), 0)
    coli = lax.broadcasted_iota(jnp.int32, (T, T), 1)
    tri = (coli < rowi).astype(jnp.bfloat16)
    cum = jnp.dot(tri, c.astype(jnp.bfloat16),
                  preferred_element_type=jnp.float32)
    tot = jnp.sum(c, axis=0, keepdims=True)
    pc = jnp.floor((tot + (BLK - 1)) * (1.0 / BLK)) * BLK
    ei = lax.broadcasted_iota(jnp.int32, (E, E), 0)
    ej = lax.broadcasted_iota(jnp.int32, (E, E), 1)
    triE = (ei < ej).astype(jnp.float32)
    base = jnp.dot(pc, triE, preferred_element_type=jnp.float32)
    ends = base + pc
    base_t = jnp.broadcast_to(base, (T, E)) + cum
    d1 = jnp.sum(oh1 * base_t, axis=-1, keepdims=True)
    d2 = jnp.sum(oh2 * base_t, axis=-1, keepdims=True)
    dest_ref[...] = jnp.concatenate([d1, d2], axis=1).astype(jnp.int32)
    gexp_ref[...] = jnp.concatenate(
        [jnp.broadcast_to(v1, (T, 16)), jnp.broadcast_to(v2, (T, 16))], axis=1)
    bv = lax.broadcasted_iota(jnp.int32, (64, E), 0).astype(jnp.float32) * BLK
    nend = jnp.sum((bv >= jnp.broadcast_to(ends, (64, E))).astype(jnp.float32),
                   axis=-1, keepdims=True)
    be = jnp.minimum(nend, float(E - 1))
    be_ref[...] = jnp.broadcast_to(be, (64, 128)).astype(jnp.int32)


def _router(x, wg):
    return pl.pallas_call(
        _router_body,
        out_shape=(
            jax.ShapeDtypeStruct((T, K), jnp.int32),
            jax.ShapeDtypeStruct((T, 2 * 16), jnp.float32),
            jax.ShapeDtypeStruct((64, 128), jnp.int32),
            jax.ShapeDtypeStruct((T, D), jnp.bfloat16),
        ),
    )(x, wg)


def _dispatch(x, toks_r, dest_r):
    mesh = plsc.VectorSubcoreMesh(core_axis_name="c", subcore_axis_name="s")

    @functools.partial(
        pl.kernel, mesh=mesh,
        out_type=jax.ShapeDtypeStruct((R, D // 2), jnp.int32),
        scratch_types=[
            pltpu.VMEM((NSH, CH), jnp.int32),
            pltpu.VMEM((NSH, CH), jnp.int32),
            pltpu.VMEM((CH, D // 2), jnp.int32),
            pltpu.VMEM((CH, D // 2), jnp.int32),
            pltpu.SemaphoreType.DMA,
            pltpu.SemaphoreType.DMA,
        ],
    )
    def k(x_hbm, toks_hbm, dest_hbm, xs_hbm, tok_v, dst_v, rows0, rows1,
          gsem, ssem):
        wid = lax.axis_index("s") * 2 + lax.axis_index("c")
        pltpu.sync_copy(toks_hbm.at[wid], tok_v)
        pltpu.sync_copy(dest_hbm.at[wid], dst_v)
        bufs = (rows0, rows1)
        gops = [None] * NSH
        sops = [None] * NSH
        gops[0] = pltpu.async_copy(x_hbm.at[tok_v.at[0]], rows0, gsem)
        gops[1] = pltpu.async_copy(x_hbm.at[tok_v.at[1]], rows1, gsem)
        for s in range(NSH):
            gops[s].wait()
            sops[s] = pltpu.async_copy(bufs[s % 2], xs_hbm.at[dst_v.at[s]],
                                       ssem)
            if s + 2 < NSH:
                sops[s].wait()
                gops[s + 2] = pltpu.async_copy(
                    x_hbm.at[tok_v.at[s + 2]], bufs[s % 2], gsem)
        sops[NSH - 2].wait()
        sops[NSH - 1].wait()

    return k(x, toks_r, dest_r)


def _ffn_body(be_ref, xs_ref, w1_ref, b1_ref, w2_ref, b2_ref, ys_ref):
    del be_ref
    xb = xs_ref[...]
    h = jnp.dot(xb, w1_ref[0], preferred_element_type=jnp.float32) + b1_ref[0]
    h = 0.5 * h * (1.0 + lax.erf(h * 0.7071067811865476))
    y = jnp.dot(h.astype(jnp.bfloat16), w2_ref[0],
                preferred_element_type=jnp.float32) + b2_ref[0]
    ys_ref[...] = y


def _ffn(be_pad, xs, W1, b1, W2, b2):
    grid_spec = pltpu.PrefetchScalarGridSpec(
        num_scalar_prefetch=1,
        grid=(NB,),
        in_specs=[
            pl.BlockSpec((BLK, D), lambda b, be_s: (b, 0)),
            pl.BlockSpec((1, D, H), lambda b, be_s: (be_s[b, 0], 0, 0)),
            pl.BlockSpec((1, 1, H), lambda b, be_s: (be_s[b, 0], 0, 0)),
            pl.BlockSpec((1, H, D), lambda b, be_s: (be_s[b, 0], 0, 0)),
            pl.BlockSpec((1, 1, D), lambda b, be_s: (be_s[b, 0], 0, 0)),
        ],
        out_specs=pl.BlockSpec((BLK, D), lambda b, be_s: (b, 0)),
    )
    return pl.pallas_call(
        _ffn_body,
        grid_spec=grid_spec,
        out_shape=jax.ShapeDtypeStruct((R, D), jnp.float32),
    )(be_pad, xs, W1.astype(jnp.bfloat16), b1.reshape(E, 1, H),
      W2.astype(jnp.bfloat16), b2.reshape(E, 1, D))


def _combine(ys, dest_r, gexp):
    mesh = plsc.VectorSubcoreMesh(core_axis_name="c", subcore_axis_name="s")

    @functools.partial(
        pl.kernel, mesh=mesh,
        out_type=jax.ShapeDtypeStruct((T, D), jnp.float32),
        scratch_types=[
            pltpu.VMEM((NSH, CH), jnp.int32),
            pltpu.VMEM((TPW, 32), jnp.float32),
            pltpu.VMEM((CH, D), jnp.float32),
            pltpu.VMEM((CH, D), jnp.float32),
            pltpu.VMEM((TPS, D), jnp.float32),
            pltpu.SemaphoreType.DMA,
        ],
    )
    def k(ys_hbm, dest_hbm, gexp_hbm, out_hbm, dst_v, g_v, rows0, rows1,
          out_v, sem):
        wid = lax.axis_index("s") * 2 + lax.axis_index("c")
        pltpu.sync_copy(dest_hbm.at[wid], dst_v)
        pltpu.sync_copy(gexp_hbm.at[pl.ds(wid * TPW, TPW)], g_v)
        bufs = (rows0, rows1)
        gops = [None] * NSH
        gops[0] = pltpu.async_copy(ys_hbm.at[dst_v.at[0]], rows0, sem)
        gops[1] = pltpu.async_copy(ys_hbm.at[dst_v.at[1]], rows1, sem)
        for s in range(NSH):
            gops[s].wait()
            rows_v = bufs[s % 2]

            def body(j, _, s=s, rows_v=rows_v):
                tok = s * TPS + j
                g0 = g_v[tok, pl.ds(0, 16)]
                g1 = g_v[tok, pl.ds(16, 16)]
                for cidx in range(D // 16):
                    sl = pl.ds(cidx * 16, 16)
                    out_v[j, sl] = g0 * rows_v[2 * j, sl] + g1 * rows_v[2 * j + 1, sl]
                return 0

            lax.fori_loop(0, TPS, body, 0)
            pltpu.sync_copy(out_v, out_hbm.at[pl.ds(wid * TPW + s * TPS, TPS)])
            if s + 2 < NSH:
                gops[s + 2] = pltpu.async_copy(
                    ys_hbm.at[dst_v.at[s + 2]], rows_v, sem)

    return k(ys, dest_r, gexp)


_TOKS = np.ascontiguousarray(
    (np.arange(T * K, dtype=np.int32) // K).reshape(NW, NSH, CH))


def kernel(x, W_gate, W1, b1, W2, b2):
    dest, gexp, be_pad, xb16 = _router(x, W_gate)
    dest_r = dest.reshape(NW, NSH, CH)
    xb_i32 = lax.bitcast_convert_type(xb16.reshape(T, D // 2, 2), jnp.int32)
    xs_i32 = _dispatch(xb_i32, _TOKS, dest_r)
    xs = lax.bitcast_convert_type(xs_i32, jnp.bfloat16).reshape(R, D)
    ys = _ffn(be_pad, xs, W1, b1, W2, b2)
    return _combine(ys, dest_r, gexp)


# speedup vs baseline: 2.4152x; 2.4152x over previous
import functools

import jax
import jax.numpy as jnp
import numpy as np
from jax import lax
from jax.experimental import pallas as pl
from jax.experimental.pallas import tpu as pltpu
from jax.experimental.pallas import tpu_sc as plsc

T, D, H, E, K = 2048, 1024, 512, 8, 2
BLK = 128
NB = T * K // BLK + E
R = NB * BLK
NW = 32
PAIRS_PER_W = T * K // NW
CH = 32
NSH = PAIRS_PER_W // CH
TPW = T // NW
TPS = CH // K


def _router_body(x_ref, wg_ref, dest_ref, gexp_ref, be_ref):
    x = x_ref[...]
    logits = jnp.dot(x, wg_ref[...], preferred_element_type=jnp.float32)
    m = jnp.max(logits, axis=-1, keepdims=True)
    ex = jnp.exp(logits - m)
    p = ex / jnp.sum(ex, axis=-1, keepdims=True)
    lane = lax.broadcasted_iota(jnp.int32, (T, E), 1)
    v1 = jnp.max(p, axis=-1, keepdims=True)
    i1 = jnp.min(jnp.where(p == v1, lane, E), axis=-1, keepdims=True)
    p2 = jnp.where(lane == i1, -jnp.inf, p)
    v2 = jnp.max(p2, axis=-1, keepdims=True)
    i2 = jnp.min(jnp.where(p2 == v2, lane, E), axis=-1, keepdims=True)
    oh1 = (lane == i1).astype(jnp.float32)
    oh2 = (lane == i2).astype(jnp.float32)
    c = oh1 + oh2
    rowi = lax.broadcasted_iota(jnp.int32, (T, T), 0)
    coli = lax.broadcasted_iota(jnp.int32, (T, T), 1)
    tri = (coli < rowi).astype(jnp.bfloat16)
    cum = jnp.dot(tri, c.astype(jnp.bfloat16),
                  preferred_element_type=jnp.float32)
    tot = jnp.sum(c, axis=0, keepdims=True)
    pc = jnp.floor((tot + (BLK - 1)) * (1.0 / BLK)) * BLK
    ei = lax.broadcasted_iota(jnp.int32, (E, E), 0)
    ej = lax.broadcasted_iota(jnp.int32, (E, E), 1)
    triE = (ei < ej).astype(jnp.float32)
    base = jnp.dot(pc, triE, preferred_element_type=jnp.float32)
    ends = base + pc
    base_t = jnp.broadcast_to(base, (T, E)) + cum
    d1 = jnp.sum(oh1 * base_t, axis=-1, keepdims=True)
    d2 = jnp.sum(oh2 * base_t, axis=-1, keepdims=True)
    dest_ref[...] = jnp.concatenate([d1, d2], axis=1).astype(jnp.int32)
    gexp_ref[...] = jnp.concatenate(
        [jnp.broadcast_to(v1, (T, 16)), jnp.broadcast_to(v2, (T, 16))], axis=1)
    bv = lax.broadcasted_iota(jnp.int32, (64, E), 0).astype(jnp.float32) * BLK
    nend = jnp.sum((bv >= jnp.broadcast_to(ends, (64, E))).astype(jnp.float32),
                   axis=-1, keepdims=True)
    be = jnp.minimum(nend, float(E - 1))
    be_ref[...] = jnp.broadcast_to(be, (64, 128)).astype(jnp.int32)


def _router(x, wg):
    return pl.pallas_call(
        _router_body,
        out_shape=(
            jax.ShapeDtypeStruct((T, K), jnp.int32),
            jax.ShapeDtypeStruct((T, 2 * 16), jnp.float32),
            jax.ShapeDtypeStruct((64, 128), jnp.int32),
        ),
    )(x, wg)


def _dispatch(x, toks_r, dest_r):
    mesh = plsc.VectorSubcoreMesh(core_axis_name="c", subcore_axis_name="s")

    @functools.partial(
        pl.kernel, mesh=mesh,
        out_type=jax.ShapeDtypeStruct((R, D), jnp.float32),
        scratch_types=[
            pltpu.VMEM((NSH, CH), jnp.int32),
            pltpu.VMEM((NSH, CH), jnp.int32),
            pltpu.VMEM((CH, D), jnp.float32),
            pltpu.VMEM((CH, D), jnp.float32),
            pltpu.SemaphoreType.DMA,
            pltpu.SemaphoreType.DMA,
        ],
    )
    def k(x_hbm, toks_hbm, dest_hbm, xs_hbm, tok_v, dst_v, rows0, rows1,
          gsem, ssem):
        wid = lax.axis_index("s") * 2 + lax.axis_index("c")
        pltpu.sync_copy(toks_hbm.at[wid], tok_v)
        pltpu.sync_copy(dest_hbm.at[wid], dst_v)
        bufs = (rows0, rows1)
        gops = [None] * NSH
        sops = [None] * NSH
        gops[0] = pltpu.async_copy(x_hbm.at[tok_v.at[0]], rows0, gsem)
        gops[1] = pltpu.async_copy(x_hbm.at[tok_v.at[1]], rows1, gsem)
        for s in range(NSH):
            gops[s].wait()
            sops[s] = pltpu.async_copy(bufs[s % 2], xs_hbm.at[dst_v.at[s]],
                                       ssem)
            if s + 2 < NSH:
                sops[s].wait()
                gops[s + 2] = pltpu.async_copy(
                    x_hbm.at[tok_v.at[s + 2]], bufs[s % 2], gsem)
        sops[NSH - 2].wait()
        sops[NSH - 1].wait()

    return k(x, toks_r, dest_r)


def _ffn_body(be_ref, xs_ref, w1_ref, b1_ref, w2_ref, b2_ref, ys_ref):
    del be_ref
    xb = xs_ref[...].astype(jnp.bfloat16)
    h = jnp.dot(xb, w1_ref[0].astype(jnp.bfloat16),
                preferred_element_type=jnp.float32) + b1_ref[0]
    h = 0.5 * h * (1.0 + lax.erf(h * 0.7071067811865476))
    y = jnp.dot(h.astype(jnp.bfloat16), w2_ref[0].astype(jnp.bfloat16),
                preferred_element_type=jnp.float32) + b2_ref[0]
    ys_ref[...] = y


def _ffn(be_pad, xs, W1, b1, W2, b2):
    grid_spec = pltpu.PrefetchScalarGridSpec(
        num_scalar_prefetch=1,
        grid=(NB,),
        in_specs=[
            pl.BlockSpec((BLK, D), lambda b, be_s: (b, 0)),
            pl.BlockSpec((1, D, H), lambda b, be_s: (be_s[b, 0], 0, 0)),
            pl.BlockSpec((1, 1, H), lambda b, be_s: (be_s[b, 0], 0, 0)),
            pl.BlockSpec((1, H, D), lambda b, be_s: (be_s[b, 0], 0, 0)),
            pl.BlockSpec((1, 1, D), lambda b, be_s: (be_s[b, 0], 0, 0)),
        ],
        out_specs=pl.BlockSpec((BLK, D), lambda b, be_s: (b, 0)),
    )
    return pl.pallas_call(
        _ffn_body,
        grid_spec=grid_spec,
        out_shape=jax.ShapeDtypeStruct((R, D), jnp.float32),
    )(be_pad, xs, W1, b1.reshape(E, 1, H), W2, b2.reshape(E, 1, D))


def _combine(ys, dest_r, gexp):
    mesh = plsc.VectorSubcoreMesh(core_axis_name="c", subcore_axis_name="s")

    @functools.partial(
        pl.kernel, mesh=mesh,
        out_type=jax.ShapeDtypeStruct((T, D), jnp.float32),
        scratch_types=[
            pltpu.VMEM((NSH, CH), jnp.int32),
            pltpu.VMEM((TPW, 32), jnp.float32),
            pltpu.VMEM((CH, D), jnp.float32),
            pltpu.VMEM((CH, D), jnp.float32),
            pltpu.VMEM((TPS, D), jnp.float32),
            pltpu.SemaphoreType.DMA,
        ],
    )
    def k(ys_hbm, dest_hbm, gexp_hbm, out_hbm, dst_v, g_v, rows0, rows1,
          out_v, sem):
        wid = lax.axis_index("s") * 2 + lax.axis_index("c")
        pltpu.sync_copy(dest_hbm.at[wid], dst_v)
        pltpu.sync_copy(gexp_hbm.at[pl.ds(wid * TPW, TPW)], g_v)
        bufs = (rows0, rows1)
        gops = [None] * NSH
        gops[0] = pltpu.async_copy(ys_hbm.at[dst_v.at[0]], rows0, sem)
        gops[1] = pltpu.async_copy(ys_hbm.at[dst_v.at[1]], rows1, sem)
        for s in range(NSH):
            gops[s].wait()
            rows_v = bufs[s % 2]

            @plsc.parallel_loop(0, TPS)
            def body(j, s=s, rows_v=rows_v):
                tok = s * TPS + j
                g0 = g_v[tok, pl.ds(0, 16)]
                g1 = g_v[tok, pl.ds(16, 16)]
                for cidx in range(D // 16):
                    sl = pl.ds(cidx * 16, 16)
                    out_v[j, sl] = g0 * rows_v[2 * j, sl] + g1 * rows_v[2 * j + 1, sl]

            pltpu.sync_copy(out_v, out_hbm.at[pl.ds(wid * TPW + s * TPS, TPS)])
            if s + 2 < NSH:
                gops[s + 2] = pltpu.async_copy(
                    ys_hbm.at[dst_v.at[s + 2]], rows_v, sem)

    return k(ys, dest_r, gexp)


_TOKS = np.ascontiguousarray(
    (np.arange(T * K, dtype=np.int32) // K).reshape(NW, NSH, CH))


def kernel(x, W_gate, W1, b1, W2, b2):
    dest, gexp, be_pad = _router(x, W_gate)
    dest_r = dest.reshape(NW, NSH, CH)
    xs = _dispatch(x, _TOKS, dest_r)
    ys = _ffn(be_pad, xs, W1, b1, W2, b2)
    return _combine(ys, dest_r, gexp)


# speedup vs baseline: 2.6267x; 1.0876x over previous
import functools

import jax
import jax.numpy as jnp
import numpy as np
from jax import lax
from jax.experimental import pallas as pl
from jax.experimental.pallas import tpu as pltpu
from jax.experimental.pallas import tpu_sc as plsc

T, D, H, E, K = 2048, 1024, 512, 8, 2
BLK = 128
NB = T * K // BLK + E
R = NB * BLK
NW = 32
PAIRS_PER_W = T * K // NW
CH = 16
NSH = PAIRS_PER_W // CH
TPW = T // NW
CH4 = 32
NSH4 = PAIRS_PER_W // CH4
TPS = CH4 // K
IMASK = 0xFFFF
GMASK = -65536


def _router_body(x_ref, wg_ref, dest_ref, gf_ref, be_ref):
    x = x_ref[...]
    logits = jnp.dot(x, wg_ref[...], preferred_element_type=jnp.float32)
    m = jnp.max(logits, axis=-1, keepdims=True)
    ex = jnp.exp(logits - m)
    p = ex / jnp.sum(ex, axis=-1, keepdims=True)
    lane = lax.broadcasted_iota(jnp.int32, (T, E), 1)
    v1 = jnp.max(p, axis=-1, keepdims=True)
    i1 = jnp.min(jnp.where(p == v1, lane, E), axis=-1, keepdims=True)
    p2 = jnp.where(lane == i1, -jnp.inf, p)
    v2 = jnp.max(p2, axis=-1, keepdims=True)
    i2 = jnp.min(jnp.where(p2 == v2, lane, E), axis=-1, keepdims=True)
    oh1 = (lane == i1).astype(jnp.float32)
    oh2 = (lane == i2).astype(jnp.float32)
    c = oh1 + oh2
    rowi = lax.broadcasted_iota(jnp.int32, (T, T), 0)
    coli = lax.broadcasted_iota(jnp.int32, (T, T), 1)
    tri = (coli < rowi).astype(jnp.bfloat16)
    cum = jnp.dot(tri, c.astype(jnp.bfloat16),
                  preferred_element_type=jnp.float32)
    tot = jnp.sum(c, axis=0, keepdims=True)
    pc = jnp.floor((tot + (BLK - 1)) * (1.0 / BLK)) * BLK
    ei = lax.broadcasted_iota(jnp.int32, (E, E), 0)
    ej = lax.broadcasted_iota(jnp.int32, (E, E), 1)
    triE = (ei < ej).astype(jnp.float32)
    base = jnp.dot(pc, triE, preferred_element_type=jnp.float32)
    ends = base + pc
    base_t = jnp.broadcast_to(base, (T, E)) + cum
    d1 = jnp.sum(oh1 * base_t, axis=-1, keepdims=True).astype(jnp.int32)
    d2 = jnp.sum(oh2 * base_t, axis=-1, keepdims=True).astype(jnp.int32)
    dest_ref[...] = jnp.concatenate(
        [jnp.transpose(d1), jnp.transpose(d2)], axis=0)
    gf_ref[...] = jnp.concatenate(
        [jnp.transpose(v1), jnp.transpose(v2)], axis=0)
    bv = lax.broadcasted_iota(jnp.int32, (64, E), 0).astype(jnp.float32) * BLK
    nend = jnp.sum((bv >= jnp.broadcast_to(ends, (64, E))).astype(jnp.float32),
                   axis=-1, keepdims=True)
    be = jnp.minimum(nend, float(E - 1))
    be_ref[...] = jnp.broadcast_to(be, (64, 128)).astype(jnp.int32)


def _router(x, wg):
    return pl.pallas_call(
        _router_body,
        out_shape=(
            jax.ShapeDtypeStruct((K, T), jnp.int32),
            jax.ShapeDtypeStruct((K, T), jnp.float32),
            jax.ShapeDtypeStruct((64, 128), jnp.int32),
        ),
    )(x, wg)


def _dispatch(x, destp):
    mesh = plsc.VectorSubcoreMesh(core_axis_name="c", subcore_axis_name="s")

    @functools.partial(
        pl.kernel, mesh=mesh,
        out_type=jax.ShapeDtypeStruct((R, D), jnp.float32),
        scratch_types=[
            pltpu.VMEM((PAIRS_PER_W,), jnp.int32),
            pltpu.VMEM((CH, D), jnp.float32),
            pltpu.VMEM((CH, D), jnp.float32),
            pltpu.VMEM((CH, D), jnp.float32),
            pltpu.SemaphoreType.DMA,
            pltpu.SemaphoreType.DMA,
        ],
    )
    def k(x_hbm, dest_hbm, xs_hbm, dst_v, rows0, rows1, rows2, gsem, ssem):
        wid = lax.axis_index("s") * 2 + lax.axis_index("c")
        tbase = wid * TPW
        pltpu.sync_copy(dest_hbm.at[0, pl.ds(tbase, TPW)],
                        dst_v.at[pl.ds(0, TPW)])
        pltpu.sync_copy(dest_hbm.at[1, pl.ds(tbase, TPW)],
                        dst_v.at[pl.ds(TPW, TPW)])
        bufs = (rows0, rows1, rows2)
        iota = lax.iota(jnp.int32, 16)

        def issue_g(s):
            tokv = tbase + (s % (TPW // CH)) * CH + iota
            return pltpu.async_copy(x_hbm.at[tokv], bufs[s % 3], gsem)

        gops = [None] * NSH
        sops = [None] * NSH
        for s in range(3):
            gops[s] = issue_g(s)
        for s in range(NSH):
            gops[s].wait()
            dvec = dst_v[pl.ds(s * CH, CH)]
            sops[s] = pltpu.async_copy(bufs[s % 3], xs_hbm.at[dvec], ssem)
            if s + 3 < NSH:
                sops[s].wait()
                gops[s + 3] = issue_g(s + 3)
        for s in range(NSH - 3, NSH):
            sops[s].wait()

    return k(x, destp)


def _ffn_body(be_ref, xs_ref, w1_ref, b1_ref, w2_ref, b2_ref, ys_ref):
    del be_ref
    xb = xs_ref[...].astype(jnp.bfloat16)
    h = jnp.dot(xb, w1_ref[0].astype(jnp.bfloat16),
                preferred_element_type=jnp.float32) + b1_ref[0]
    h = 0.5 * h * (1.0 + lax.erf(h * 0.7071067811865476))
    y = jnp.dot(h.astype(jnp.bfloat16), w2_ref[0].astype(jnp.bfloat16),
                preferred_element_type=jnp.float32) + b2_ref[0]
    ys_ref[...] = y


def _ffn(be_pad, xs, W1, b1, W2, b2):
    grid_spec = pltpu.PrefetchScalarGridSpec(
        num_scalar_prefetch=1,
        grid=(NB,),
        in_specs=[
            pl.BlockSpec((BLK, D), lambda b, be_s: (b, 0)),
            pl.BlockSpec((1, D, H), lambda b, be_s: (be_s[b, 0], 0, 0)),
            pl.BlockSpec((1, 1, H), lambda b, be_s: (be_s[b, 0], 0, 0)),
            pl.BlockSpec((1, H, D), lambda b, be_s: (be_s[b, 0], 0, 0)),
            pl.BlockSpec((1, 1, D), lambda b, be_s: (be_s[b, 0], 0, 0)),
        ],
        out_specs=pl.BlockSpec((BLK, D), lambda b, be_s: (b, 0)),
    )
    return pl.pallas_call(
        _ffn_body,
        grid_spec=grid_spec,
        out_shape=jax.ShapeDtypeStruct((R, D), jnp.float32),
    )(be_pad, xs, W1, b1.reshape(E, 1, H), W2, b2.reshape(E, 1, D))


def _bcast_lane(v, lane):
    idx = jnp.full((16, 1), lane, jnp.int32)
    dn = lax.GatherDimensionNumbers(
        offset_dims=(), collapsed_slice_dims=(0,), start_index_map=(0,))
    return lax.gather(v, idx, dn, (1,),
                      mode=lax.GatherScatterMode.PROMISE_IN_BOUNDS)


def _combine(ys, destp, gf):
    mesh = plsc.VectorSubcoreMesh(core_axis_name="c", subcore_axis_name="s")

    @functools.partial(
        pl.kernel, mesh=mesh,
        out_type=jax.ShapeDtypeStruct((T, D), jnp.float32),
        scratch_types=[
            pltpu.VMEM((PAIRS_PER_W,), jnp.int32),
            pltpu.VMEM((PAIRS_PER_W + 16,), jnp.float32),
            pltpu.VMEM((CH4, D), jnp.float32),
            pltpu.VMEM((CH4, D), jnp.float32),
            pltpu.VMEM((CH4, D), jnp.float32),
            pltpu.VMEM((TPS, D), jnp.float32),
            pltpu.SemaphoreType.DMA,
            pltpu.SemaphoreType.DMA,
        ],
    )
    def k(ys_hbm, dest_hbm, gf_hbm, out_hbm, dst_v, g_v, rows0, rows1, rows2,
          out_v, gsem, osem):
        wid = lax.axis_index("s") * 2 + lax.axis_index("c")
        tbase = wid * TPW
        pltpu.sync_copy(dest_hbm.at[0, pl.ds(tbase, TPW)],
                        dst_v.at[pl.ds(0, TPW)])
        pltpu.sync_copy(dest_hbm.at[1, pl.ds(tbase, TPW)],
                        dst_v.at[pl.ds(TPW, TPW)])
        pltpu.sync_copy(gf_hbm.at[0, pl.ds(tbase, TPW)],
                        g_v.at[pl.ds(0, TPW)])
        pltpu.sync_copy(gf_hbm.at[1, pl.ds(tbase, TPW)],
                        g_v.at[pl.ds(TPW, TPW)])
        bufs = (rows0, rows1, rows2)

        def issue_g(s):
            buf = bufs[s % 3]
            iv0 = dst_v[pl.ds(s * TPS, TPS)]
            iv1 = dst_v[pl.ds(TPW + s * TPS, TPS)]
            return (
                pltpu.async_copy(ys_hbm.at[iv0], buf.at[pl.ds(0, TPS)], gsem),
                pltpu.async_copy(ys_hbm.at[iv1], buf.at[pl.ds(TPS, TPS)], gsem),
            )

        gops = [None] * NSH4
        oops = [None] * NSH4
        for s in range(3):
            gops[s] = issue_g(s)
        for s in range(NSH4):
            gops[s][0].wait()
            gops[s][1].wait()
            if s >= 1:
                oops[s - 1].wait()
            rows_v = bufs[s % 3]
            gs0 = g_v[pl.ds(s * TPS, TPS)]
            gs1 = g_v[pl.ds(TPW + s * TPS, TPS)]

            @plsc.parallel_loop(0, TPS)
            def body(j, rows_v=rows_v, gs0=gs0, gs1=gs1):
                g0 = _bcast_lane(gs0, j)
                g1 = _bcast_lane(gs1, j)
                for cidx in range(D // 16):
                    sl = pl.ds(cidx * 16, 16)
                    out_v[j, sl] = g0 * rows_v[j, sl] + g1 * rows_v[TPS + j, sl]

            if s + 3 < NSH4:
                gops[s + 3] = issue_g(s + 3)
            oops[s] = pltpu.async_copy(
                out_v, out_hbm.at[pl.ds(tbase + s * TPS, TPS)], osem)
        oops[NSH4 - 1].wait()

    return k(ys, destp, gf)


def kernel(x, W_gate, W1, b1, W2, b2):
    destp, gf, be_pad = _router(x, W_gate)
    xs = _dispatch(x, destp)
    ys = _ffn(be_pad, xs, W1, b1, W2, b2)
    return _combine(ys, destp, gf)


# speedup vs baseline: 2.6564x; 1.0113x over previous
import functools

import jax
import jax.numpy as jnp
import numpy as np
from jax import lax
from jax.experimental import pallas as pl
from jax.experimental.pallas import tpu as pltpu
from jax.experimental.pallas import tpu_sc as plsc

T, D, H, E, K = 2048, 1024, 512, 8, 2
BLK = 128
NB = T * K // BLK + E
R = NB * BLK
NW = 32
PAIRS_PER_W = T * K // NW
CH = 16
NSH = PAIRS_PER_W // CH
TPW = T // NW
CH4 = 32
NSH4 = PAIRS_PER_W // CH4
TPS = CH4 // K
IMASK = 0xFFFF
GMASK = -65536


def _router_body(x_ref, wg_ref, dest_ref, gf_ref, be_ref):
    x = x_ref[...]
    logits = jnp.dot(x, wg_ref[...], preferred_element_type=jnp.float32)
    m = jnp.max(logits, axis=-1, keepdims=True)
    ex = jnp.exp(logits - m)
    p = ex / jnp.sum(ex, axis=-1, keepdims=True)
    lane = lax.broadcasted_iota(jnp.int32, (T, E), 1)
    v1 = jnp.max(p, axis=-1, keepdims=True)
    i1 = jnp.min(jnp.where(p == v1, lane, E), axis=-1, keepdims=True)
    p2 = jnp.where(lane == i1, -jnp.inf, p)
    v2 = jnp.max(p2, axis=-1, keepdims=True)
    i2 = jnp.min(jnp.where(p2 == v2, lane, E), axis=-1, keepdims=True)
    oh1 = (lane == i1).astype(jnp.float32)
    oh2 = (lane == i2).astype(jnp.float32)
    c = oh1 + oh2
    rowi = lax.broadcasted_iota(jnp.int32, (T, T), 0)
    coli = lax.broadcasted_iota(jnp.int32, (T, T), 1)
    tri = (coli < rowi).astype(jnp.bfloat16)
    cum = jnp.dot(tri, c.astype(jnp.bfloat16),
                  preferred_element_type=jnp.float32)
    tot = jnp.sum(c, axis=0, keepdims=True)
    pc = jnp.floor((tot + (BLK - 1)) * (1.0 / BLK)) * BLK
    ei = lax.broadcasted_iota(jnp.int32, (E, E), 0)
    ej = lax.broadcasted_iota(jnp.int32, (E, E), 1)
    triE = (ei < ej).astype(jnp.float32)
    base = jnp.dot(pc, triE, preferred_element_type=jnp.float32)
    ends = base + pc
    base_t = jnp.broadcast_to(base, (T, E)) + cum
    d1 = jnp.sum(oh1 * base_t, axis=-1, keepdims=True).astype(jnp.int32)
    d2 = jnp.sum(oh2 * base_t, axis=-1, keepdims=True).astype(jnp.int32)
    dest_ref[...] = jnp.concatenate(
        [jnp.transpose(d1), jnp.transpose(d2)], axis=0)
    gf_ref[...] = jnp.concatenate(
        [jnp.transpose(v1), jnp.transpose(v2)], axis=0)
    bv = lax.broadcasted_iota(jnp.int32, (64, E), 0).astype(jnp.float32) * BLK
    nend = jnp.sum((bv >= jnp.broadcast_to(ends, (64, E))).astype(jnp.float32),
                   axis=-1, keepdims=True)
    be = jnp.minimum(nend, float(E - 1))
    be_ref[...] = jnp.broadcast_to(be, (64, 128)).astype(jnp.int32)


def _router(x, wg):
    return pl.pallas_call(
        _router_body,
        out_shape=(
            jax.ShapeDtypeStruct((K, T), jnp.int32),
            jax.ShapeDtypeStruct((K, T), jnp.float32),
            jax.ShapeDtypeStruct((64, 128), jnp.int32),
        ),
    )(x, wg)


def _dispatch(x, destp):
    mesh = plsc.VectorSubcoreMesh(core_axis_name="c", subcore_axis_name="s")

    @functools.partial(
        pl.kernel, mesh=mesh,
        out_type=jax.ShapeDtypeStruct((R, D), jnp.float32),
        scratch_types=[
            pltpu.VMEM((PAIRS_PER_W,), jnp.int32),
            pltpu.VMEM((CH, D), jnp.float32),
            pltpu.VMEM((CH, D), jnp.float32),
            pltpu.VMEM((CH, D), jnp.float32),
            pltpu.SemaphoreType.DMA,
            pltpu.SemaphoreType.DMA,
        ],
    )
    def k(x_hbm, dest_hbm, xs_hbm, dst_v, rows0, rows1, rows2, gsem, ssem):
        wid = lax.axis_index("s") * 2 + lax.axis_index("c")
        tbase = wid * TPW
        pltpu.sync_copy(dest_hbm.at[0, pl.ds(tbase, TPW)],
                        dst_v.at[pl.ds(0, TPW)])
        pltpu.sync_copy(dest_hbm.at[1, pl.ds(tbase, TPW)],
                        dst_v.at[pl.ds(TPW, TPW)])
        bufs = (rows0, rows1, rows2)
        iota = lax.iota(jnp.int32, 16)

        def issue_g(s):
            tokv = tbase + (s % (TPW // CH)) * CH + iota
            return pltpu.async_copy(x_hbm.at[tokv], bufs[s % 3], gsem)

        gops = [None] * NSH
        sops = [None] * NSH
        for s in range(3):
            gops[s] = issue_g(s)
        for s in range(NSH):
            gops[s].wait()
            dvec = dst_v[pl.ds(s * CH, CH)]
            sops[s] = pltpu.async_copy(bufs[s % 3], xs_hbm.at[dvec], ssem)
            if s + 3 < NSH:
                sops[s].wait()
                gops[s + 3] = issue_g(s + 3)
        for s in range(NSH - 3, NSH):
            sops[s].wait()

    return k(x, destp)


def _ffn_body(be_ref, xs_ref, w1_ref, b1_ref, w2_ref, b2_ref, ys_ref):
    e = be_ref[pl.program_id(0), 0]
    xb = xs_ref[...].astype(jnp.bfloat16)
    h = jnp.dot(xb, w1_ref[e].astype(jnp.bfloat16),
                preferred_element_type=jnp.float32) + b1_ref[e]
    h = 0.5 * h * (1.0 + lax.erf(h * 0.7071067811865476))
    y = jnp.dot(h.astype(jnp.bfloat16), w2_ref[e].astype(jnp.bfloat16),
                preferred_element_type=jnp.float32) + b2_ref[e]
    ys_ref[...] = y


def _ffn(be_pad, xs, W1, b1, W2, b2):
    grid_spec = pltpu.PrefetchScalarGridSpec(
        num_scalar_prefetch=1,
        grid=(NB,),
        in_specs=[
            pl.BlockSpec((BLK, D), lambda b, be_s: (b, 0)),
            pl.BlockSpec((E, D, H), lambda b, be_s: (0, 0, 0)),
            pl.BlockSpec((E, 1, H), lambda b, be_s: (0, 0, 0)),
            pl.BlockSpec((E, H, D), lambda b, be_s: (0, 0, 0)),
            pl.BlockSpec((E, 1, D), lambda b, be_s: (0, 0, 0)),
        ],
        out_specs=pl.BlockSpec((BLK, D), lambda b, be_s: (b, 0)),
    )
    return pl.pallas_call(
        _ffn_body,
        grid_spec=grid_spec,
        out_shape=jax.ShapeDtypeStruct((R, D), jnp.float32),
    )(be_pad, xs, W1, b1.reshape(E, 1, H), W2, b2.reshape(E, 1, D))


def _bcast_lane(v, lane):
    idx = jnp.full((16, 1), lane, jnp.int32)
    dn = lax.GatherDimensionNumbers(
        offset_dims=(), collapsed_slice_dims=(0,), start_index_map=(0,))
    return lax.gather(v, idx, dn, (1,),
                      mode=lax.GatherScatterMode.PROMISE_IN_BOUNDS)


def _combine(ys, destp, gf):
    mesh = plsc.VectorSubcoreMesh(core_axis_name="c", subcore_axis_name="s")

    @functools.partial(
        pl.kernel, mesh=mesh,
        out_type=jax.ShapeDtypeStruct((T, D), jnp.float32),
        scratch_types=[
            pltpu.VMEM((PAIRS_PER_W,), jnp.int32),
            pltpu.VMEM((PAIRS_PER_W + 16,), jnp.float32),
            pltpu.VMEM((CH4, D), jnp.float32),
            pltpu.VMEM((CH4, D), jnp.float32),
            pltpu.VMEM((CH4, D), jnp.float32),
            pltpu.VMEM((TPS, D), jnp.float32),
            pltpu.SemaphoreType.DMA,
            pltpu.SemaphoreType.DMA,
        ],
    )
    def k(ys_hbm, dest_hbm, gf_hbm, out_hbm, dst_v, g_v, rows0, rows1, rows2,
          out_v, gsem, osem):
        wid = lax.axis_index("s") * 2 + lax.axis_index("c")
        tbase = wid * TPW
        pltpu.sync_copy(dest_hbm.at[0, pl.ds(tbase, TPW)],
                        dst_v.at[pl.ds(0, TPW)])
        pltpu.sync_copy(dest_hbm.at[1, pl.ds(tbase, TPW)],
                        dst_v.at[pl.ds(TPW, TPW)])
        pltpu.sync_copy(gf_hbm.at[0, pl.ds(tbase, TPW)],
                        g_v.at[pl.ds(0, TPW)])
        pltpu.sync_copy(gf_hbm.at[1, pl.ds(tbase, TPW)],
                        g_v.at[pl.ds(TPW, TPW)])
        bufs = (rows0, rows1, rows2)

        def issue_g(s):
            buf = bufs[s % 3]
            iv0 = dst_v[pl.ds(s * TPS, TPS)]
            iv1 = dst_v[pl.ds(TPW + s * TPS, TPS)]
            return (
                pltpu.async_copy(ys_hbm.at[iv0], buf.at[pl.ds(0, TPS)], gsem),
                pltpu.async_copy(ys_hbm.at[iv1], buf.at[pl.ds(TPS, TPS)], gsem),
            )

        gops = [None] * NSH4
        oops = [None] * NSH4
        for s in range(3):
            gops[s] = issue_g(s)
        for s in range(NSH4):
            gops[s][0].wait()
            gops[s][1].wait()
            if s >= 1:
                oops[s - 1].wait()
            rows_v = bufs[s % 3]
            gs0 = g_v[pl.ds(s * TPS, TPS)]
            gs1 = g_v[pl.ds(TPW + s * TPS, TPS)]

            @plsc.parallel_loop(0, TPS)
            def body(j, rows_v=rows_v, gs0=gs0, gs1=gs1):
                g0 = _bcast_lane(gs0, j)
                g1 = _bcast_lane(gs1, j)
                for cidx in range(D // 16):
                    sl = pl.ds(cidx * 16, 16)
                    out_v[j, sl] = g0 * rows_v[j, sl] + g1 * rows_v[TPS + j, sl]

            if s + 3 < NSH4:
                gops[s + 3] = issue_g(s + 3)
            oops[s] = pltpu.async_copy(
                out_v, out_hbm.at[pl.ds(tbase + s * TPS, TPS)], osem)
        oops[NSH4 - 1].wait()

    return k(ys, destp, gf)


def kernel(x, W_gate, W1, b1, W2, b2):
    destp, gf, be_pad = _router(x, W_gate)
    xs = _dispatch(x, destp)
    ys = _ffn(be_pad, xs, W1, b1, W2, b2)
    return _combine(ys, destp, gf)


# speedup vs baseline: 2.7487x; 1.0348x over previous
import functools

import jax
import jax.numpy as jnp
import numpy as np
from jax import lax
from jax.experimental import pallas as pl
from jax.experimental.pallas import tpu as pltpu
from jax.experimental.pallas import tpu_sc as plsc

T, D, H, E, K = 2048, 1024, 512, 8, 2
BLK = 128
NB = T * K // BLK + E
R = NB * BLK
NW = 32
PAIRS_PER_W = T * K // NW
CH = 16
NSH = PAIRS_PER_W // CH
TPW = T // NW
CH4 = 32
NSH4 = PAIRS_PER_W // CH4
TPS = CH4 // K
IMASK = 0xFFFF
GMASK = -65536


def _router_body(x_ref, wg_ref, dest_ref, gf_ref, be_ref):
    x = x_ref[...]
    logits = jnp.dot(x, wg_ref[...], preferred_element_type=jnp.float32)
    m = jnp.max(logits, axis=-1, keepdims=True)
    ex = jnp.exp(logits - m)
    p = ex / jnp.sum(ex, axis=-1, keepdims=True)
    lane = lax.broadcasted_iota(jnp.int32, (T, E), 1)
    v1 = jnp.max(p, axis=-1, keepdims=True)
    i1 = jnp.min(jnp.where(p == v1, lane, E), axis=-1, keepdims=True)
    p2 = jnp.where(lane == i1, -jnp.inf, p)
    v2 = jnp.max(p2, axis=-1, keepdims=True)
    i2 = jnp.min(jnp.where(p2 == v2, lane, E), axis=-1, keepdims=True)
    oh1 = (lane == i1).astype(jnp.float32)
    oh2 = (lane == i2).astype(jnp.float32)
    c = oh1 + oh2
    rowi = lax.broadcasted_iota(jnp.int32, (T, T), 0)
    coli = lax.broadcasted_iota(jnp.int32, (T, T), 1)
    tri = (coli < rowi).astype(jnp.bfloat16)
    cum = jnp.dot(tri, c.astype(jnp.bfloat16),
                  preferred_element_type=jnp.float32)
    tot = jnp.sum(c, axis=0, keepdims=True)
    pc = jnp.floor((tot + (BLK - 1)) * (1.0 / BLK)) * BLK
    ei = lax.broadcasted_iota(jnp.int32, (E, E), 0)
    ej = lax.broadcasted_iota(jnp.int32, (E, E), 1)
    triE = (ei < ej).astype(jnp.float32)
    base = jnp.dot(pc, triE, preferred_element_type=jnp.float32)
    ends = base + pc
    base_t = jnp.broadcast_to(base, (T, E)) + cum
    d1 = jnp.sum(oh1 * base_t, axis=-1, keepdims=True).astype(jnp.int32)
    d2 = jnp.sum(oh2 * base_t, axis=-1, keepdims=True).astype(jnp.int32)
    dest_ref[...] = jnp.concatenate(
        [jnp.transpose(d1), jnp.transpose(d2)], axis=0)
    gf_ref[...] = jnp.concatenate(
        [jnp.transpose(v1), jnp.transpose(v2)], axis=0)
    bv = lax.broadcasted_iota(jnp.int32, (64, E), 0).astype(jnp.float32) * BLK
    nend = jnp.sum((bv >= jnp.broadcast_to(ends, (64, E))).astype(jnp.float32),
                   axis=-1, keepdims=True)
    be = jnp.minimum(nend, float(E - 1))
    nv = ends[:, E - 1:E] * (1.0 / BLK)
    rowi64 = lax.broadcasted_iota(jnp.int32, (64, 1), 0)
    be = jnp.where(rowi64 == 63, nv, be)
    be_ref[...] = jnp.broadcast_to(be, (64, 128)).astype(jnp.int32)


def _router(x, wg):
    return pl.pallas_call(
        _router_body,
        out_shape=(
            jax.ShapeDtypeStruct((K, T), jnp.int32),
            jax.ShapeDtypeStruct((K, T), jnp.float32),
            jax.ShapeDtypeStruct((64, 128), jnp.int32),
        ),
    )(x, wg)


def _dispatch(x, destp):
    mesh = plsc.VectorSubcoreMesh(core_axis_name="c", subcore_axis_name="s")

    @functools.partial(
        pl.kernel, mesh=mesh,
        out_type=jax.ShapeDtypeStruct((R, D), jnp.float32),
        scratch_types=[
            pltpu.VMEM((PAIRS_PER_W,), jnp.int32),
            pltpu.VMEM((CH, D), jnp.float32),
            pltpu.VMEM((CH, D), jnp.float32),
            pltpu.VMEM((CH, D), jnp.float32),
            pltpu.SemaphoreType.DMA,
            pltpu.SemaphoreType.DMA,
        ],
    )
    def k(x_hbm, dest_hbm, xs_hbm, dst_v, rows0, rows1, rows2, gsem, ssem):
        wid = lax.axis_index("s") * 2 + lax.axis_index("c")
        tbase = wid * TPW
        pltpu.sync_copy(dest_hbm.at[0, pl.ds(tbase, TPW)],
                        dst_v.at[pl.ds(0, TPW)])
        pltpu.sync_copy(dest_hbm.at[1, pl.ds(tbase, TPW)],
                        dst_v.at[pl.ds(TPW, TPW)])
        bufs = (rows0, rows1, rows2)
        iota = lax.iota(jnp.int32, 16)

        def issue_g(s):
            tokv = tbase + (s % (TPW // CH)) * CH + iota
            return pltpu.async_copy(x_hbm.at[tokv], bufs[s % 3], gsem)

        gops = [None] * NSH
        sops = [None] * NSH
        for s in range(3):
            gops[s] = issue_g(s)
        for s in range(NSH):
            gops[s].wait()
            dvec = dst_v[pl.ds(s * CH, CH)]
            sops[s] = pltpu.async_copy(bufs[s % 3], xs_hbm.at[dvec], ssem)
            if s + 3 < NSH:
                sops[s].wait()
                gops[s + 3] = issue_g(s + 3)
        for s in range(NSH - 3, NSH):
            sops[s].wait()

    return k(x, destp)


def _ffn_body(be_ref, xs_ref, w1_ref, b1_ref, w2_ref, b2_ref, ys_ref):
    b = pl.program_id(0)
    nv = be_ref[63, 0]

    @pl.when(b < nv)
    def _():
        e = be_ref[b, 0]
        xb = xs_ref[...].astype(jnp.bfloat16)
        h = jnp.dot(xb, w1_ref[e].astype(jnp.bfloat16),
                    preferred_element_type=jnp.float32) + b1_ref[e]
        h = 0.5 * h * (1.0 + lax.erf(h * 0.7071067811865476))
        y = jnp.dot(h.astype(jnp.bfloat16), w2_ref[e].astype(jnp.bfloat16),
                    preferred_element_type=jnp.float32) + b2_ref[e]
        ys_ref[...] = y


def _row_idx(b, be_s):
    return (jnp.minimum(b, be_s[63, 0] - 1), 0)


def _ffn(be_pad, xs, W1, b1, W2, b2):
    grid_spec = pltpu.PrefetchScalarGridSpec(
        num_scalar_prefetch=1,
        grid=(NB,),
        in_specs=[
            pl.BlockSpec((BLK, D), _row_idx),
            pl.BlockSpec((E, D, H), lambda b, be_s: (0, 0, 0)),
            pl.BlockSpec((E, 1, H), lambda b, be_s: (0, 0, 0)),
            pl.BlockSpec((E, H, D), lambda b, be_s: (0, 0, 0)),
            pl.BlockSpec((E, 1, D), lambda b, be_s: (0, 0, 0)),
        ],
        out_specs=pl.BlockSpec((BLK, D), _row_idx),
    )
    return pl.pallas_call(
        _ffn_body,
        grid_spec=grid_spec,
        out_shape=jax.ShapeDtypeStruct((R, D), jnp.float32),
    )(be_pad, xs, W1, b1.reshape(E, 1, H), W2, b2.reshape(E, 1, D))


def _bcast_lane(v, lane):
    idx = jnp.full((16, 1), lane, jnp.int32)
    dn = lax.GatherDimensionNumbers(
        offset_dims=(), collapsed_slice_dims=(0,), start_index_map=(0,))
    return lax.gather(v, idx, dn, (1,),
                      mode=lax.GatherScatterMode.PROMISE_IN_BOUNDS)


def _combine(ys, destp, gf):
    mesh = plsc.VectorSubcoreMesh(core_axis_name="c", subcore_axis_name="s")

    @functools.partial(
        pl.kernel, mesh=mesh,
        out_type=jax.ShapeDtypeStruct((T, D), jnp.float32),
        scratch_types=[
            pltpu.VMEM((PAIRS_PER_W,), jnp.int32),
            pltpu.VMEM((PAIRS_PER_W + 16,), jnp.float32),
            pltpu.VMEM((CH4, D), jnp.float32),
            pltpu.VMEM((CH4, D), jnp.float32),
            pltpu.VMEM((CH4, D), jnp.float32),
            pltpu.VMEM((TPS, D), jnp.float32),
            pltpu.SemaphoreType.DMA,
            pltpu.SemaphoreType.DMA,
        ],
    )
    def k(ys_hbm, dest_hbm, gf_hbm, out_hbm, dst_v, g_v, rows0, rows1, rows2,
          out_v, gsem, osem):
        wid = lax.axis_index("s") * 2 + lax.axis_index("c")
        tbase = wid * TPW
        pltpu.sync_copy(dest_hbm.at[0, pl.ds(tbase, TPW)],
                        dst_v.at[pl.ds(0, TPW)])
        pltpu.sync_copy(dest_hbm.at[1, pl.ds(tbase, TPW)],
                        dst_v.at[pl.ds(TPW, TPW)])
        pltpu.sync_copy(gf_hbm.at[0, pl.ds(tbase, TPW)],
                        g_v.at[pl.ds(0, TPW)])
        pltpu.sync_copy(gf_hbm.at[1, pl.ds(tbase, TPW)],
                        g_v.at[pl.ds(TPW, TPW)])
        bufs = (rows0, rows1, rows2)

        def issue_g(s):
            buf = bufs[s % 3]
            iv0 = dst_v[pl.ds(s * TPS, TPS)]
            iv1 = dst_v[pl.ds(TPW + s * TPS, TPS)]
            return (
                pltpu.async_copy(ys_hbm.at[iv0], buf.at[pl.ds(0, TPS)], gsem),
                pltpu.async_copy(ys_hbm.at[iv1], buf.at[pl.ds(TPS, TPS)], gsem),
            )

        gops = [None] * NSH4
        oops = [None] * NSH4
        for s in range(3):
            gops[s] = issue_g(s)
        for s in range(NSH4):
            gops[s][0].wait()
            gops[s][1].wait()
            if s >= 1:
                oops[s - 1].wait()
            rows_v = bufs[s % 3]
            gs0 = g_v[pl.ds(s * TPS, TPS)]
            gs1 = g_v[pl.ds(TPW + s * TPS, TPS)]

            @plsc.parallel_loop(0, TPS)
            def body(j, rows_v=rows_v, gs0=gs0, gs1=gs1):
                g0 = _bcast_lane(gs0, j)
                g1 = _bcast_lane(gs1, j)
                for cidx in range(D // 16):
                    sl = pl.ds(cidx * 16, 16)
                    out_v[j, sl] = g0 * rows_v[j, sl] + g1 * rows_v[TPS + j, sl]

            if s + 3 < NSH4:
                gops[s + 3] = issue_g(s + 3)
            oops[s] = pltpu.async_copy(
                out_v, out_hbm.at[pl.ds(tbase + s * TPS, TPS)], osem)
        oops[NSH4 - 1].wait()

    return k(ys, destp, gf)


def kernel(x, W_gate, W1, b1, W2, b2):
    destp, gf, be_pad = _router(x, W_gate)
    xs = _dispatch(x, destp)
    ys = _ffn(be_pad, xs, W1, b1, W2, b2)
    return _combine(ys, destp, gf)
